# baseline (speedup 1.0000x reference)
import functools

import jax
import jax.numpy as jnp
from jax.experimental import pallas as pl
from jax.experimental.pallas import tpu as pltpu

COMMIT_SCALE = 0.0001


def _fused_kernel(x_ref, means_ref, dists_ref, loss_ref, msq_ref):
    i = pl.program_id(0)
    j = pl.program_id(1)

    x = x_ref[...]
    m = means_ref[...]

    @pl.when(j == 0)
    def _msq():
        msq_ref[...] = jnp.sum(m * m, axis=-1)[None, :]

    sq = jnp.sum(x * x, axis=-1, keepdims=True)
    rinv = jax.lax.rsqrt(jnp.maximum(sq, 1e-24))
    xn = x * rinv

    d = jax.lax.dot_general(
        xn, m, (((1,), (1,)), ((), ())), preferred_element_type=jnp.float32
    )
    dists_ref[...] = d

    dmax = jnp.max(d, axis=-1, keepdims=True)
    msq_sel = jnp.max(
        jnp.where(d == dmax, msq_ref[...], -jnp.inf), axis=-1
    )
    xnsq = sq[:, 0] * (rinv[:, 0] * rinv[:, 0])
    partial = jnp.sum(xnsq - 2.0 * dmax[:, 0] + msq_sel)

    @pl.when((i == 0) & (j == 0))
    def _init():
        loss_ref[0, 0] = 0.0

    loss_ref[0, 0] += partial


def kernel(x, means):
    B, H, L, D = x.shape
    Hm, C, Dm = means.shape
    xr = x.reshape(B * H, L, D)

    LT = 512
    grid = (B * H, L // LT)

    dists, loss = pl.pallas_call(
        _fused_kernel,
        grid=grid,
        in_specs=[
            pl.BlockSpec((None, LT, D), lambda i, j: (i, j, 0)),
            pl.BlockSpec((None, C, Dm), lambda i, j: (i % Hm, 0, 0)),
        ],
        out_specs=[
            pl.BlockSpec((None, LT, C), lambda i, j: (i, j, 0)),
            pl.BlockSpec(memory_space=pltpu.SMEM),
        ],
        out_shape=[
            jax.ShapeDtypeStruct((B * H, L, C), jnp.float32),
            jax.ShapeDtypeStruct((1, 1), jnp.float32),
        ],
        scratch_shapes=[pltpu.VMEM((1, C), jnp.float32)],
    )(xr, means)

    loss_scalar = loss[0, 0] * (COMMIT_SCALE / (B * H * L * D))
    return (dists.reshape(B, H, L, C), loss_scalar)


# speedup vs baseline: 1.3306x; 1.3306x over previous
import functools

import jax
import jax.numpy as jnp
from jax.experimental import pallas as pl
from jax.experimental.pallas import tpu as pltpu

COMMIT_SCALE = 0.0001


def _fused_kernel(x_ref, means_ref, dists_ref, loss_ref, msq_ref):
    i = pl.program_id(0)
    j = pl.program_id(1)

    x = x_ref[...]
    m = means_ref[...]

    @pl.when(j == 0)
    def _msq():
        msq_ref[...] = jnp.sum(m * m, axis=-1)[None, :]

    sq = jnp.sum(x * x, axis=-1, keepdims=True)
    rinv = jax.lax.rsqrt(jnp.maximum(sq, 1e-24))
    xn = x * rinv

    d = jax.lax.dot_general(
        xn, m, (((1,), (1,)), ((), ())), preferred_element_type=jnp.float32
    )
    dists_ref[...] = d

    dmax = jnp.max(d, axis=-1, keepdims=True)
    msq_sel = jnp.max(
        jnp.where(d == dmax, msq_ref[...], -jnp.inf), axis=-1
    )
    xnsq = sq[:, 0] * (rinv[:, 0] * rinv[:, 0])
    partial = jnp.sum(xnsq - 2.0 * dmax[:, 0] + msq_sel)

    @pl.when((i == 0) & (j == 0))
    def _init():
        loss_ref[0, 0] = 0.0

    loss_ref[0, 0] += partial


def kernel(x, means):
    B, H, L, D = x.shape
    Hm, C, Dm = means.shape
    xr = x.reshape(B * H, L, D)

    LT = 1024
    grid = (B * H, L // LT)

    dists, loss = pl.pallas_call(
        _fused_kernel,
        grid=grid,
        in_specs=[
            pl.BlockSpec((None, LT, D), lambda i, j: (i, j, 0)),
            pl.BlockSpec((None, C, Dm), lambda i, j: (i % Hm, 0, 0)),
        ],
        out_specs=[
            pl.BlockSpec((None, LT, C), lambda i, j: (i, j, 0)),
            pl.BlockSpec(memory_space=pltpu.SMEM),
        ],
        out_shape=[
            jax.ShapeDtypeStruct((B * H, L, C), jnp.float32),
            jax.ShapeDtypeStruct((1, 1), jnp.float32),
        ],
        scratch_shapes=[pltpu.VMEM((1, C), jnp.float32)],
    )(xr, means)

    loss_scalar = loss[0, 0] * (COMMIT_SCALE / (B * H * L * D))
    return (dists.reshape(B, H, L, C), loss_scalar)


# speedup vs baseline: 1.6727x; 1.2571x over previous
import functools

import jax
import jax.numpy as jnp
from jax.experimental import pallas as pl
from jax.experimental.pallas import tpu as pltpu

COMMIT_SCALE = 0.0001


def _fused_kernel(x_ref, means_ref, dists_ref, loss_ref, msq_ref):
    i = pl.program_id(0)
    j = pl.program_id(1)

    x = x_ref[...]
    m = means_ref[...]

    @pl.when(j == 0)
    def _msq():
        msq_ref[...] = jnp.sum(m * m, axis=-1)[None, :]

    sq = jnp.sum(x * x, axis=-1, keepdims=True)
    rinv = jax.lax.rsqrt(jnp.maximum(sq, 1e-24))
    xn = x * rinv

    d = jax.lax.dot_general(
        xn, m, (((1,), (1,)), ((), ())), preferred_element_type=jnp.float32
    )
    dists_ref[...] = d

    dmax = jnp.max(d, axis=-1, keepdims=True)
    msq_sel = jnp.max(
        jnp.where(d == dmax, msq_ref[...], -jnp.inf), axis=-1
    )
    xnsq = sq[:, 0] * (rinv[:, 0] * rinv[:, 0])
    partial = jnp.sum(xnsq - 2.0 * dmax[:, 0] + msq_sel)

    @pl.when((i == 0) & (j == 0))
    def _init():
        loss_ref[0, 0] = 0.0

    loss_ref[0, 0] += partial


def kernel(x, means):
    B, H, L, D = x.shape
    Hm, C, Dm = means.shape
    xr = x.reshape(B * H, L, D)

    LT = 2048
    grid = (B * H, L // LT)

    dists, loss = pl.pallas_call(
        _fused_kernel,
        grid=grid,
        in_specs=[
            pl.BlockSpec((None, LT, D), lambda i, j: (i, j, 0)),
            pl.BlockSpec((None, C, Dm), lambda i, j: (i % Hm, 0, 0)),
        ],
        out_specs=[
            pl.BlockSpec((None, LT, C), lambda i, j: (i, j, 0)),
            pl.BlockSpec(memory_space=pltpu.SMEM),
        ],
        out_shape=[
            jax.ShapeDtypeStruct((B * H, L, C), jnp.float32),
            jax.ShapeDtypeStruct((1, 1), jnp.float32),
        ],
        scratch_shapes=[pltpu.VMEM((1, C), jnp.float32)],
    )(xr, means)

    loss_scalar = loss[0, 0] * (COMMIT_SCALE / (B * H * L * D))
    return (dists.reshape(B, H, L, C), loss_scalar)


# speedup vs baseline: 2.0014x; 1.1965x over previous
import functools

import jax
import jax.numpy as jnp
from jax.experimental import pallas as pl
from jax.experimental.pallas import tpu as pltpu

COMMIT_SCALE = 0.0001


def _fused_kernel(x_ref, means_ref, dists_ref, loss_ref, msq_ref):
    i = pl.program_id(0)
    j = pl.program_id(1)

    x = x_ref[...]
    m = means_ref[...]

    @pl.when(j == 0)
    def _msq():
        msq_ref[...] = jnp.sum(m * m, axis=-1)[None, :]

    sq = jnp.sum(x * x, axis=-1, keepdims=True)
    rinv = jax.lax.rsqrt(jnp.maximum(sq, 1e-24))
    xn = x * rinv

    d = jax.lax.dot_general(
        xn, m, (((1,), (1,)), ((), ())), preferred_element_type=jnp.float32
    )
    dists_ref[...] = d

    dmax = jnp.max(d, axis=-1, keepdims=True)
    msq_sel = jnp.max(
        jnp.where(d == dmax, msq_ref[...], -jnp.inf), axis=-1
    )
    xnsq = sq[:, 0] * (rinv[:, 0] * rinv[:, 0])
    partial = jnp.sum(xnsq - 2.0 * dmax[:, 0] + msq_sel)

    @pl.when((i == 0) & (j == 0))
    def _init():
        loss_ref[0, 0] = 0.0

    loss_ref[0, 0] += partial


def kernel(x, means):
    B, H, L, D = x.shape
    Hm, C, Dm = means.shape
    xr = x.reshape(B * H, L, D)

    LT = 4096
    grid = (B * H, L // LT)

    dists, loss = pl.pallas_call(
        _fused_kernel,
        grid=grid,
        in_specs=[
            pl.BlockSpec((None, LT, D), lambda i, j: (i, j, 0)),
            pl.BlockSpec((None, C, Dm), lambda i, j: (i % Hm, 0, 0)),
        ],
        out_specs=[
            pl.BlockSpec((None, LT, C), lambda i, j: (i, j, 0)),
            pl.BlockSpec(memory_space=pltpu.SMEM),
        ],
        out_shape=[
            jax.ShapeDtypeStruct((B * H, L, C), jnp.float32),
            jax.ShapeDtypeStruct((1, 1), jnp.float32),
        ],
        scratch_shapes=[pltpu.VMEM((1, C), jnp.float32)],
    )(xr, means)

    loss_scalar = loss[0, 0] * (COMMIT_SCALE / (B * H * L * D))
    return (dists.reshape(B, H, L, C), loss_scalar)
